# baseline (device time: 93504 ns/iter reference)
import jax
import jax.numpy as jnp
from jax import lax
from jax.experimental import pallas as pl
from jax.experimental.pallas import tpu as pltpu

N_DEV = 4
T_CORR = 96

BF = jnp.bfloat16


def kernel(x, A, B, C):
    Bb, S, D = x.shape
    N = A.shape[1]

    def body(x_ref, a_ref, b_ref, c_ref, out_ref,
             hbuf, hrecv, xt_s, yt_s, bt_s, ct_s, da_s,
             send_sem, recv_sem):
        my = lax.axis_index("i")
        left = lax.rem(my + N_DEV - 1, N_DEV)
        right = lax.rem(my + 1, N_DEV)

        barrier = pltpu.get_barrier_semaphore()
        pl.semaphore_signal(barrier, inc=1, device_id=(left,),
                            device_id_type=pl.DeviceIdType.MESH)
        pl.semaphore_signal(barrier, inc=1, device_id=(right,),
                            device_id_type=pl.DeviceIdType.MESH)
        pl.semaphore_wait(barrier, 2)

        a_t = a_ref[:, :].T
        da_s[:] = (jnp.exp(a_t)[:, None, :]
                   + jnp.zeros((N, Bb, D), jnp.float32)).astype(BF)
        xt_s[:] = jnp.swapaxes(x_ref[:], 0, 1).astype(BF)
        bt_s[:] = jnp.swapaxes(b_ref[:], 0, 1).astype(BF)
        ct_s[:] = jnp.swapaxes(c_ref[:], 0, 1).astype(BF)

        def step(t, h):
            x_t = xt_s[pl.ds(t, 1)]
            bg = bt_s[pl.ds(t, 1)][0]
            cg = ct_s[pl.ds(t, 1)][0]
            da = da_s[:]
            hs = []
            for n in range(N):
                hs.append(h[n] * da[n] + x_t[0] * bg[:, n:n + 1])
            h = jnp.stack(hs, axis=0)
            y_t = hs[0] * cg[:, 0:1]
            for n in range(1, N):
                y_t = y_t + hs[n] * cg[:, n:n + 1]
            yt_s[pl.ds(t, 1)] = y_t[None]
            return h

        h0 = jnp.zeros((N, Bb, D), BF)
        h_final = lax.fori_loop(0, S, step, h0)

        hbuf[:] = h_final
        rdma = pltpu.make_async_remote_copy(
            src_ref=hbuf,
            dst_ref=hrecv,
            send_sem=send_sem,
            recv_sem=recv_sem,
            device_id=(right,),
            device_id_type=pl.DeviceIdType.MESH,
        )
        rdma.start()
        rdma.wait()

        @pl.when(my != 0)
        def _():
            def corr_step(t, g):
                da = da_s[:]
                cg = ct_s[pl.ds(t, 1)][0]
                gs = []
                for n in range(N):
                    gs.append(g[n] * da[n])
                g = jnp.stack(gs, axis=0)
                y_t = gs[0] * cg[:, 0:1]
                for n in range(1, N):
                    y_t = y_t + gs[n] * cg[:, n:n + 1]
                yt_s[pl.ds(t, 1)] = yt_s[pl.ds(t, 1)] + y_t[None]
                return g

            lax.fori_loop(0, T_CORR, corr_step, hrecv[:])

        out_ref[:] = jnp.swapaxes(yt_s[:], 0, 1).astype(jnp.float32)

    return pl.pallas_call(
        body,
        out_shape=jax.ShapeDtypeStruct((Bb, S, D), jnp.float32),
        in_specs=[
            pl.BlockSpec(memory_space=pltpu.VMEM),
            pl.BlockSpec(memory_space=pltpu.VMEM),
            pl.BlockSpec(memory_space=pltpu.VMEM),
            pl.BlockSpec(memory_space=pltpu.VMEM),
        ],
        out_specs=pl.BlockSpec(memory_space=pltpu.VMEM),
        scratch_shapes=[
            pltpu.VMEM((N, Bb, D), BF),
            pltpu.VMEM((N, Bb, D), BF),
            pltpu.VMEM((S, Bb, D), BF),
            pltpu.VMEM((S, Bb, D), BF),
            pltpu.VMEM((S, Bb, N), BF),
            pltpu.VMEM((S, Bb, N), BF),
            pltpu.VMEM((N, Bb, D), BF),
            pltpu.SemaphoreType.DMA,
            pltpu.SemaphoreType.DMA,
        ],
        compiler_params=pltpu.CompilerParams(collective_id=0),
    )(x, A, B, C)
